# baseline (speedup 1.0000x reference)
import numpy as np

import jax
import jax.numpy as jnp
from jax.experimental import pallas as pl
from jax.experimental.pallas import tpu as pltpu

FLOW = 16
PASS = 8
NBINS = 64
T = FLOW - PASS
TN = T * NBINS
LANES = 128
CHUNKS = TN // LANES
SUB = LANES // T

_PERM = np.empty(TN, np.int32)
for _j2 in range(TN):
    _c, _r = divmod(_j2, LANES)
    _PERM[_j2] = (_r // SUB) * NBINS + (SUB * _c + _r % SUB)


def _fused_body(x_ref, w1_ref, b1_ref, w2_ref, b2_ref, o_ref):
    x = x_ref[...]
    xA = x[:, :PASS]
    xB = x[:, PASS:FLOW]
    jac = x[:, FLOW:FLOW + 1]

    h = jnp.tanh(
        jnp.dot(xA, w1_ref[...], preferred_element_type=jnp.float32)
        + b1_ref[...])
    logits = (jnp.dot(h, w2_ref[...], preferred_element_type=jnp.float32)
              + b2_ref[...])
    q = jax.nn.softplus(logits)

    col = jax.lax.broadcasted_iota(jnp.int32, (1, TN), 1)
    lane = jnp.bitwise_and(col, LANES - 1)
    kf = (jnp.left_shift(jnp.right_shift(col, 7), 4)
          + jnp.bitwise_and(lane, SUB - 1)).astype(jnp.float32)
    grp = jnp.right_shift(lane, 4)
    row = jax.lax.broadcasted_iota(jnp.int32, (T, TN), 0)
    bmat = (row == grp).astype(jnp.float32)

    ab = xB * NBINS
    binf = jnp.floor(ab)
    frac = ab - binf
    bcast = jnp.dot(jnp.concatenate([binf, frac], axis=0), bmat,
                    preferred_element_type=jnp.float32)
    nb = x.shape[0]
    binb = bcast[:nb]
    fracb = bcast[nb:]
    w = jnp.clip(binb + fracb - kf, 0.0, 1.0)
    eq = (binb == kf).astype(jnp.float32)

    def fold(v):
        return (v[:, :LANES] + v[:, LANES:2 * LANES]
                + v[:, 2 * LANES:3 * LANES] + v[:, 3 * LANES:])

    stacked = jnp.concatenate(
        [fold(q), fold(q * w), fold(q * eq)], axis=0)
    lrow = jax.lax.broadcasted_iota(jnp.int32, (LANES, T), 0)
    lcol = jax.lax.broadcasted_iota(jnp.int32, (LANES, T), 1)
    gmat = (jnp.right_shift(lrow, 4) == lcol).astype(jnp.float32)
    red = jnp.dot(stacked, gmat,
                  preferred_element_type=jnp.float32)
    s = red[:nb]
    num = red[nb:2 * nb]
    qb = red[2 * nb:]

    rcp = 1.0 / s
    cdf = num * rcp
    qf = qb * (rcp * NBINS)
    for t in range(T):
        jac = jac * qf[:, t:t + 1]
    o_ref[...] = jnp.concatenate([xA, cdf, jac], axis=-1)


@jax.jit
def kernel(x, W1, b1, W2, b2):
    batch = x.shape[0]
    bb = 1024
    grid = batch // bb
    b1r = b1.reshape(1, -1)
    w2p = W2[:, _PERM]
    b2r = b2[_PERM].reshape(1, -1)
    return pl.pallas_call(
        _fused_body,
        grid=(grid,),
        in_specs=[
            pl.BlockSpec((bb, FLOW + 1), lambda i: (i, 0)),
            pl.BlockSpec(W1.shape, lambda i: (0, 0)),
            pl.BlockSpec(b1r.shape, lambda i: (0, 0)),
            pl.BlockSpec(w2p.shape, lambda i: (0, 0)),
            pl.BlockSpec(b2r.shape, lambda i: (0, 0)),
        ],
        out_specs=pl.BlockSpec((bb, FLOW + 1), lambda i: (i, 0)),
        out_shape=jax.ShapeDtypeStruct((batch, FLOW + 1), jnp.float32),
        compiler_params=pltpu.CompilerParams(
            dimension_semantics=("parallel",)),
    )(x, W1, b1r, w2p, b2r)


# speedup vs baseline: 1.0955x; 1.0955x over previous
import jax
import jax.numpy as jnp
from jax.experimental import pallas as pl
from jax.experimental.pallas import tpu as pltpu

FLOW = 16
PASS = 8
NBINS = 64
T = FLOW - PASS
TN = T * NBINS


def _fused_body(x_ref, w1_ref, b1_ref, w2_ref, b2_ref, o_ref):
    x = x_ref[...]
    xA = x[:, :PASS]
    xB = x[:, PASS:FLOW]
    jac = x[:, FLOW:FLOW + 1]

    h = jnp.tanh(
        jnp.dot(xA, w1_ref[...], preferred_element_type=jnp.float32)
        + b1_ref[...])
    logits = (jnp.dot(h, w2_ref[...], preferred_element_type=jnp.float32)
              + b2_ref[...])
    q = jax.nn.softplus(logits)

    col = jax.lax.broadcasted_iota(jnp.int32, (1, TN), 1)
    kf = jnp.bitwise_and(col, NBINS - 1).astype(jnp.float32)
    grp = jnp.right_shift(col, 6)
    row = jax.lax.broadcasted_iota(jnp.int32, (T, TN), 0)
    bmat = (row == grp).astype(jnp.float32)

    ab = xB * NBINS
    binf = jnp.floor(ab)
    frac = ab - binf
    bcast = jnp.dot(jnp.concatenate([binf, frac], axis=0), bmat,
                    preferred_element_type=jnp.float32)
    nb = x.shape[0]
    binb = bcast[:nb]
    fracb = bcast[nb:]
    w = jnp.clip(binb + fracb - kf, 0.0, 1.0)
    eq = (binb == kf).astype(jnp.float32)

    stacked = jnp.concatenate([q, q * w, q * eq], axis=0)
    red = jax.lax.dot_general(
        stacked, bmat, (((1,), (1,)), ((), ())),
        preferred_element_type=jnp.float32)
    s = red[:nb]
    num = red[nb:2 * nb]
    qb = red[2 * nb:]

    rcp = 1.0 / s
    cdf = num * rcp
    qf = qb * (rcp * NBINS)
    for t in range(T):
        jac = jac * qf[:, t:t + 1]
    o_ref[...] = jnp.concatenate([xA, cdf, jac], axis=-1)


@jax.jit
def kernel(x, W1, b1, W2, b2):
    batch = x.shape[0]
    bb = 2048
    grid = batch // bb
    b1r = b1.reshape(1, -1)
    b2r = b2.reshape(1, -1)
    return pl.pallas_call(
        _fused_body,
        grid=(grid,),
        in_specs=[
            pl.BlockSpec((bb, FLOW + 1), lambda i: (i, 0)),
            pl.BlockSpec(W1.shape, lambda i: (0, 0)),
            pl.BlockSpec(b1r.shape, lambda i: (0, 0)),
            pl.BlockSpec(W2.shape, lambda i: (0, 0)),
            pl.BlockSpec(b2r.shape, lambda i: (0, 0)),
        ],
        out_specs=pl.BlockSpec((bb, FLOW + 1), lambda i: (i, 0)),
        out_shape=jax.ShapeDtypeStruct((batch, FLOW + 1), jnp.float32),
        compiler_params=pltpu.CompilerParams(
            dimension_semantics=("parallel",)),
    )(x, W1, b1r, W2, b2r)


# speedup vs baseline: 1.3463x; 1.2290x over previous
import jax
import jax.numpy as jnp
from jax.experimental import pallas as pl
from jax.experimental.pallas import tpu as pltpu

FLOW = 16
PASS = 8
NBINS = 64
T = FLOW - PASS
TN = T * NBINS


def _fused_body(x_ref, w1_ref, b1_ref, w2_ref, b2_ref, o_ref):
    x = x_ref[...]
    xA = x[:, :PASS]
    xB = x[:, PASS:FLOW]
    jac = x[:, FLOW:FLOW + 1]

    h = jnp.tanh(
        jnp.dot(xA, w1_ref[...], preferred_element_type=jnp.float32)
        + b1_ref[...])
    logits = (jnp.dot(h, w2_ref[...], preferred_element_type=jnp.float32)
              + b2_ref[...])
    q = jax.nn.softplus(logits)

    col = jax.lax.broadcasted_iota(jnp.int32, (1, TN), 1)
    kf = jnp.bitwise_and(col, NBINS - 1).astype(jnp.float32)
    grp = jnp.right_shift(col, 6)
    row = jax.lax.broadcasted_iota(jnp.int32, (T, TN), 0)
    bmat = (row == grp).astype(jnp.float32)

    ab = xB * NBINS
    binf = jnp.floor(ab)
    frac = ab - binf
    bcast = jnp.dot(jnp.concatenate([binf, frac], axis=0), bmat,
                    preferred_element_type=jnp.float32)
    nb = x.shape[0]
    binb = bcast[:nb]
    fracb = bcast[nb:]
    w = jnp.clip(binb + fracb - kf, 0.0, 1.0)
    eq = (binb == kf).astype(jnp.float32)

    stacked = jnp.concatenate([q, q * w, q * eq], axis=0)
    red = jax.lax.dot_general(
        stacked, bmat, (((1,), (1,)), ((), ())),
        preferred_element_type=jnp.float32)
    s = red[:nb]
    num = red[nb:2 * nb]
    qb = red[2 * nb:]

    rs = jax.lax.rsqrt(s)
    rcp = rs * rs
    cdf = num * rcp
    lnqf = jnp.log(qb * (rcp * NBINS))
    jac = jac * jnp.exp(jnp.sum(lnqf, axis=-1, keepdims=True))
    o_ref[...] = jnp.concatenate([xA, cdf, jac], axis=-1)


@jax.jit
def kernel(x, W1, b1, W2, b2):
    batch = x.shape[0]
    bb = 2048
    grid = batch // bb
    b1r = b1.reshape(1, -1)
    b2r = b2.reshape(1, -1)
    return pl.pallas_call(
        _fused_body,
        grid=(grid,),
        in_specs=[
            pl.BlockSpec((bb, FLOW + 1), lambda i: (i, 0)),
            pl.BlockSpec(W1.shape, lambda i: (0, 0)),
            pl.BlockSpec(b1r.shape, lambda i: (0, 0)),
            pl.BlockSpec(W2.shape, lambda i: (0, 0)),
            pl.BlockSpec(b2r.shape, lambda i: (0, 0)),
        ],
        out_specs=pl.BlockSpec((bb, FLOW + 1), lambda i: (i, 0)),
        out_shape=jax.ShapeDtypeStruct((batch, FLOW + 1), jnp.float32),
        compiler_params=pltpu.CompilerParams(
            dimension_semantics=("parallel",)),
    )(x, W1, b1r, W2, b2r)
